# baseline (speedup 1.0000x reference)
import functools

import jax
import jax.numpy as jnp
import numpy as np
from jax import lax
from jax.experimental import pallas as pl
from jax.experimental.pallas import tpu as pltpu
from jax.experimental.pallas import tpu_sc as plsc

N, E, F_IN, F_OUT, F_EDGE, HID, G = 10000, 160000, 128, 8, 16, 64, 64

NW = 32
CH = 128
EP = 163840
K = EP // (NW * CH)
EW = K * CH
MSG_W = 16
NP = 10240
ROWS_PER_TILE = NP // 16

TE_S = 32000
TE_M = 5000


def _stats_body(ea_ref, w1_ref, out_ref):
    i = pl.program_id(0)
    h = jnp.dot(ea_ref[...], w1_ref[...], preferred_element_type=jnp.float32)
    s1 = jnp.sum(h, axis=0, keepdims=True)
    s2 = jnp.sum(h * h, axis=0, keepdims=True)
    blk = jnp.concatenate([s1, s2], axis=0)

    @pl.when(i == 0)
    def _():
        out_ref[...] = jnp.zeros_like(out_ref)

    out_ref[...] += blk


_stats_call = pl.pallas_call(
    _stats_body,
    grid=(E // TE_S,),
    in_specs=[
        pl.BlockSpec((TE_S, F_EDGE), lambda i: (i, 0)),
        pl.BlockSpec((F_EDGE, HID), lambda i: (0, 0)),
    ],
    out_specs=pl.BlockSpec((2, HID), lambda i: (0, 0)),
    out_shape=jax.ShapeDtypeStruct((2, HID), jnp.float32),
)


NB = 4


def _gather_body(x_hbm, src_hbm, xj_hbm, idx_v, rows_v, sems):
    cid = lax.axis_index("c")
    sid = lax.axis_index("s")
    wid = sid * 2 + cid
    pltpu.sync_copy(src_hbm.at[wid], idx_v)
    base = wid * EW

    for b in range(NB):
        pltpu.async_copy(x_hbm.at[idx_v.at[b]], rows_v.at[b], sems.at[b])

    def group(g, carry):
        for b in range(NB):
            c = g * NB + b
            pltpu.make_async_copy(x_hbm.at[idx_v.at[c]], rows_v.at[b],
                                  sems.at[b]).wait()
            pltpu.sync_copy(rows_v.at[b], xj_hbm.at[pl.ds(base + c * CH, CH)])
            cn = c + NB

            @pl.when(cn < K)
            def _():
                pltpu.async_copy(x_hbm.at[idx_v.at[cn]], rows_v.at[b],
                                 sems.at[b])
        return carry

    lax.fori_loop(0, K // NB, group, 0)


def _make_gather():
    mesh = plsc.VectorSubcoreMesh(core_axis_name="c", subcore_axis_name="s")
    return pl.kernel(
        _gather_body,
        out_type=pltpu.HBM((EP, F_IN), jnp.float32),
        mesh=mesh,
        scratch_types=[
            pltpu.VMEM((K, CH), jnp.int32),
            pltpu.VMEM((NB, CH, F_IN), jnp.float32),
            pltpu.SemaphoreType.DMA((NB,)),
        ],
    )


def _msg_body(stats_ref, ea_ref, xj_ref, w1_ref, g_ref, b_ref, w2q_ref,
              b2m_ref, r_ref, out_ref):
    stats = stats_ref[...]
    mean = stats[0:1, :] * (1.0 / E)
    ex2 = stats[1:2, :] * (1.0 / E)
    var = ex2 - mean * mean
    inv = lax.rsqrt(var + 1e-5)
    scale = g_ref[...] * inv
    shift = b_ref[...] - mean * scale

    h0 = jnp.dot(ea_ref[...], w1_ref[...], preferred_element_type=jnp.float32)
    h = jnp.maximum(h0 * scale + shift, 0.0)

    xj = xj_ref[...].astype(jnp.bfloat16)
    u = jnp.dot(xj, w2q_ref[...], preferred_element_type=jnp.float32)
    h8 = jnp.concatenate([h] * F_OUT, axis=1)
    msg = jnp.dot(h8 * u, r_ref[...], preferred_element_type=jnp.float32)
    msg = msg + jnp.dot(xj, b2m_ref[...], preferred_element_type=jnp.float32)

    out_ref[...] = jnp.concatenate(
        [msg, jnp.zeros((TE_M, MSG_W - F_OUT), jnp.float32)], axis=1)


_msg_call = pl.pallas_call(
    _msg_body,
    grid=(E // TE_M,),
    in_specs=[
        pl.BlockSpec((2, HID), lambda i: (0, 0)),
        pl.BlockSpec((TE_M, F_EDGE), lambda i: (i, 0)),
        pl.BlockSpec((TE_M, F_IN), lambda i: (i, 0)),
        pl.BlockSpec((F_EDGE, HID), lambda i: (0, 0)),
        pl.BlockSpec((1, HID), lambda i: (0, 0)),
        pl.BlockSpec((1, HID), lambda i: (0, 0)),
        pl.BlockSpec((F_IN, F_OUT * HID), lambda i: (0, 0)),
        pl.BlockSpec((F_IN, F_OUT), lambda i: (0, 0)),
        pl.BlockSpec((F_OUT * HID, F_OUT), lambda i: (0, 0)),
    ],
    out_specs=pl.BlockSpec((TE_M, MSG_W), lambda i: (i, 0)),
    out_shape=jax.ShapeDtypeStruct((EP, MSG_W), jnp.float32),
)


NBS = 2


def _scatter_body(msg_hbm, dst_hbm, zero_hbm, out_hbm, idx_v, msg_v, shared,
                  sems):
    cid = lax.axis_index("c")
    sid = lax.axis_index("s")
    wid = sid * 2 + cid

    row0 = sid * ROWS_PER_TILE
    pltpu.sync_copy(zero_hbm.at[pl.ds(row0, ROWS_PER_TILE)],
                    shared.at[pl.ds(row0, ROWS_PER_TILE)])
    plsc.subcore_barrier()

    pltpu.sync_copy(dst_hbm.at[wid], idx_v)
    base = wid * EW

    for b in range(NBS):
        pltpu.async_copy(msg_hbm.at[pl.ds(base + b * CH, CH)], msg_v.at[b],
                         sems.at[b])

    def group(g, carry):
        for b in range(NBS):
            c = g * NBS + b
            pltpu.make_async_copy(msg_hbm.at[pl.ds(base + c * CH, CH)],
                                  msg_v.at[b], sems.at[b]).wait()

            @pl.when(base + c * CH < E)
            def _():
                pltpu.sync_copy(msg_v.at[b], shared.at[idx_v.at[c]], add=True)

            cn = c + NBS

            @pl.when(cn < K)
            def _():
                pltpu.async_copy(msg_hbm.at[pl.ds(base + cn * CH, CH)],
                                 msg_v.at[b], sems.at[b])
        return carry

    lax.fori_loop(0, K // NBS, group, 0)
    plsc.subcore_barrier()

    pltpu.sync_copy(shared.at[pl.ds(row0, ROWS_PER_TILE)],
                    out_hbm.at[cid, pl.ds(row0, ROWS_PER_TILE)])


def _make_scatter():
    mesh = plsc.VectorSubcoreMesh(core_axis_name="c", subcore_axis_name="s")
    return pl.kernel(
        _scatter_body,
        out_type=pltpu.HBM((2, NP, MSG_W), jnp.float32),
        mesh=mesh,
        compiler_params=pltpu.CompilerParams(use_tc_tiling_on_sc=False),
        scratch_types=[
            pltpu.VMEM((K, CH), jnp.int32),
            pltpu.VMEM((NBS, CH, MSG_W), jnp.float32),
            pltpu.VMEM_SHARED((NP, MSG_W), jnp.float32),
            pltpu.SemaphoreType.DMA((NBS,)),
        ],
    )


def _final_body(part_ref, x_ref, rw_ref, bias_ref, batch_ref, out_ref):
    aggr = part_ref[0][:, 0:F_OUT] + part_ref[1][:, 0:F_OUT]
    out = aggr + jnp.dot(x_ref[...], rw_ref[...],
                         preferred_element_type=jnp.float32) + bias_ref[...]
    b = batch_ref[...]
    p = (lax.broadcasted_iota(jnp.int32, (G, NP), 0) == b).astype(jnp.float32)
    sums = jnp.dot(p, out, preferred_element_type=jnp.float32)
    counts = jnp.sum(p, axis=1, keepdims=True)
    out_ref[...] = sums / jnp.maximum(counts, 1.0)


_final_call = pl.pallas_call(
    _final_body,
    out_shape=jax.ShapeDtypeStruct((G, F_OUT), jnp.float32),
)


@jax.jit
def kernel(x, adj_t, edge_attr, batch, W1, b1, bn_gamma, bn_beta, W2, b2,
           root_W, bias):
    src = adj_t[0]
    dst = adj_t[1]
    pad = EP - E
    src_p = jnp.pad(src, (0, pad)).reshape(NW, K, CH)
    dst_p = jnp.pad(dst, (0, pad)).reshape(NW, K, CH)

    w2q = W2.reshape(HID, F_IN, F_OUT).transpose(1, 2, 0).reshape(F_IN, F_OUT * HID)
    b2m = b2.reshape(F_IN, F_OUT)
    r_sel = jnp.asarray(np.kron(np.eye(F_OUT, dtype=np.float32),
                                np.ones((HID, 1), dtype=np.float32)))

    stats = _stats_call(edge_attr, W1)
    xj = _make_gather()(x, src_p)
    msg = _msg_call(stats, edge_attr, xj, W1, bn_gamma.reshape(1, HID),
                    bn_beta.reshape(1, HID), w2q.astype(jnp.bfloat16),
                    b2m.astype(jnp.bfloat16), r_sel)
    parts = _make_scatter()(msg, dst_p, jnp.zeros((NP, MSG_W), jnp.float32))
    x_p = jnp.pad(x, ((0, NP - N), (0, 0)))
    batch_p = jnp.pad(batch, (0, NP - N), constant_values=G)
    return _final_call(parts, x_p, root_W, bias.reshape(1, F_OUT),
                       batch_p.reshape(1, NP))


# speedup vs baseline: 1.0353x; 1.0353x over previous
import functools

import jax
import jax.numpy as jnp
import numpy as np
from jax import lax
from jax.experimental import pallas as pl
from jax.experimental.pallas import tpu as pltpu
from jax.experimental.pallas import tpu_sc as plsc

N, E, F_IN, F_OUT, F_EDGE, HID, G = 10000, 160000, 128, 8, 16, 64, 64

NW = 32
CH = 128
EP = 163840
K = EP // (NW * CH)
EW = K * CH
MSG_W = 16
NP = 10240
ROWS_PER_TILE = NP // 16

TE_S = 16000
TE_M = 4000


def _stats_body(ea_ref, w1_ref, out_ref):
    i = pl.program_id(0)
    h = jnp.dot(ea_ref[...], w1_ref[...], preferred_element_type=jnp.float32)
    s1 = jnp.sum(h, axis=0, keepdims=True)
    s2 = jnp.sum(h * h, axis=0, keepdims=True)
    blk = jnp.concatenate([s1, s2], axis=0)

    @pl.when(i == 0)
    def _():
        out_ref[...] = jnp.zeros_like(out_ref)

    out_ref[...] += blk


_stats_call = pl.pallas_call(
    _stats_body,
    grid=(E // TE_S,),
    in_specs=[
        pl.BlockSpec((TE_S, F_EDGE), lambda i: (i, 0)),
        pl.BlockSpec((F_EDGE, HID), lambda i: (0, 0)),
    ],
    out_specs=pl.BlockSpec((2, HID), lambda i: (0, 0)),
    out_shape=jax.ShapeDtypeStruct((2, HID), jnp.float32),
)


NB = 4


def _gather_body(x_hbm, src_hbm, xj_hbm, idx_v, rows_v, sems):
    cid = lax.axis_index("c")
    sid = lax.axis_index("s")
    wid = sid * 2 + cid
    pltpu.sync_copy(src_hbm.at[wid], idx_v)
    base = wid * EW

    for b in range(NB):
        pltpu.async_copy(x_hbm.at[idx_v.at[b]], rows_v.at[b], sems.at[b])

    def group(g, carry):
        for b in range(NB):
            c = g * NB + b
            pltpu.make_async_copy(x_hbm.at[idx_v.at[c]], rows_v.at[b],
                                  sems.at[b]).wait()
            pltpu.sync_copy(rows_v.at[b], xj_hbm.at[pl.ds(base + c * CH, CH)])
            cn = c + NB

            @pl.when(cn < K)
            def _():
                pltpu.async_copy(x_hbm.at[idx_v.at[cn]], rows_v.at[b],
                                 sems.at[b])
        return carry

    lax.fori_loop(0, K // NB, group, 0)


def _make_gather():
    mesh = plsc.VectorSubcoreMesh(core_axis_name="c", subcore_axis_name="s")
    return pl.kernel(
        _gather_body,
        out_type=pltpu.HBM((EP, F_IN), jnp.float32),
        mesh=mesh,
        scratch_types=[
            pltpu.VMEM((K, CH), jnp.int32),
            pltpu.VMEM((NB, CH, F_IN), jnp.float32),
            pltpu.SemaphoreType.DMA((NB,)),
        ],
    )


def _msg_body(stats_ref, ea_ref, xj_ref, w1_ref, g_ref, b_ref, w2q_ref,
              b2m_ref, r_ref, out_ref):
    stats = stats_ref[...]
    mean = stats[0:1, :] * (1.0 / E)
    ex2 = stats[1:2, :] * (1.0 / E)
    var = ex2 - mean * mean
    inv = lax.rsqrt(var + 1e-5)
    scale = g_ref[...] * inv
    shift = b_ref[...] - mean * scale

    h0 = jnp.dot(ea_ref[...], w1_ref[...], preferred_element_type=jnp.float32)
    h = jnp.maximum(h0 * scale + shift, 0.0)

    xj = xj_ref[...].astype(jnp.bfloat16)
    u = jnp.dot(xj, w2q_ref[...], preferred_element_type=jnp.float32)
    h8 = jnp.concatenate([h] * F_OUT, axis=1)
    msg = jnp.dot(h8 * u, r_ref[...], preferred_element_type=jnp.float32)
    msg = msg + jnp.dot(xj, b2m_ref[...], preferred_element_type=jnp.float32)

    out_ref[...] = jnp.concatenate(
        [msg, jnp.zeros((TE_M, MSG_W - F_OUT), jnp.float32)], axis=1)


_msg_call = pl.pallas_call(
    _msg_body,
    grid=(E // TE_M,),
    in_specs=[
        pl.BlockSpec((2, HID), lambda i: (0, 0)),
        pl.BlockSpec((TE_M, F_EDGE), lambda i: (i, 0)),
        pl.BlockSpec((TE_M, F_IN), lambda i: (i, 0)),
        pl.BlockSpec((F_EDGE, HID), lambda i: (0, 0)),
        pl.BlockSpec((1, HID), lambda i: (0, 0)),
        pl.BlockSpec((1, HID), lambda i: (0, 0)),
        pl.BlockSpec((F_IN, F_OUT * HID), lambda i: (0, 0)),
        pl.BlockSpec((F_IN, F_OUT), lambda i: (0, 0)),
        pl.BlockSpec((F_OUT * HID, F_OUT), lambda i: (0, 0)),
    ],
    out_specs=pl.BlockSpec((TE_M, MSG_W), lambda i: (i, 0)),
    out_shape=jax.ShapeDtypeStruct((EP, MSG_W), jnp.float32),
)


NBS = 2


def _scatter_body(msg_hbm, dst_hbm, zero_hbm, out_hbm, idx_v, msg_v, shared,
                  sems):
    cid = lax.axis_index("c")
    sid = lax.axis_index("s")
    wid = sid * 2 + cid

    row0 = sid * ROWS_PER_TILE
    pltpu.sync_copy(zero_hbm.at[pl.ds(row0, ROWS_PER_TILE)],
                    shared.at[pl.ds(row0, ROWS_PER_TILE)])
    plsc.subcore_barrier()

    pltpu.sync_copy(dst_hbm.at[wid], idx_v)
    base = wid * EW

    for b in range(NBS):
        pltpu.async_copy(msg_hbm.at[pl.ds(base + b * CH, CH)], msg_v.at[b],
                         sems.at[b])

    def group(g, carry):
        for b in range(NBS):
            c = g * NBS + b
            pltpu.make_async_copy(msg_hbm.at[pl.ds(base + c * CH, CH)],
                                  msg_v.at[b], sems.at[b]).wait()

            @pl.when(base + c * CH < E)
            def _():
                pltpu.sync_copy(msg_v.at[b], shared.at[idx_v.at[c]], add=True)

            cn = c + NBS

            @pl.when(cn < K)
            def _():
                pltpu.async_copy(msg_hbm.at[pl.ds(base + cn * CH, CH)],
                                 msg_v.at[b], sems.at[b])
        return carry

    lax.fori_loop(0, K // NBS, group, 0)
    plsc.subcore_barrier()

    pltpu.sync_copy(shared.at[pl.ds(row0, ROWS_PER_TILE)],
                    out_hbm.at[cid, pl.ds(row0, ROWS_PER_TILE)])


def _make_scatter():
    mesh = plsc.VectorSubcoreMesh(core_axis_name="c", subcore_axis_name="s")
    return pl.kernel(
        _scatter_body,
        out_type=pltpu.HBM((2, NP, MSG_W), jnp.float32),
        mesh=mesh,
        compiler_params=pltpu.CompilerParams(use_tc_tiling_on_sc=False),
        scratch_types=[
            pltpu.VMEM((K, CH), jnp.int32),
            pltpu.VMEM((NBS, CH, MSG_W), jnp.float32),
            pltpu.VMEM_SHARED((NP, MSG_W), jnp.float32),
            pltpu.SemaphoreType.DMA((NBS,)),
        ],
    )


def _final_body(part_ref, x_ref, rw_ref, bias_ref, batch_ref, out_ref):
    aggr = part_ref[0][:, 0:F_OUT] + part_ref[1][:, 0:F_OUT]
    out = aggr + jnp.dot(x_ref[...], rw_ref[...],
                         preferred_element_type=jnp.float32) + bias_ref[...]
    b = batch_ref[...]
    p = (lax.broadcasted_iota(jnp.int32, (G, NP), 0) == b).astype(jnp.float32)
    sums = jnp.dot(p, out, preferred_element_type=jnp.float32)
    counts = jnp.sum(p, axis=1, keepdims=True)
    out_ref[...] = sums / jnp.maximum(counts, 1.0)


_final_call = pl.pallas_call(
    _final_body,
    out_shape=jax.ShapeDtypeStruct((G, F_OUT), jnp.float32),
)


@jax.jit
def kernel(x, adj_t, edge_attr, batch, W1, b1, bn_gamma, bn_beta, W2, b2,
           root_W, bias):
    src = adj_t[0]
    dst = adj_t[1]
    pad = EP - E
    src_p = jnp.pad(src, (0, pad)).reshape(NW, K, CH)
    dst_p = jnp.pad(dst, (0, pad)).reshape(NW, K, CH)

    w2q = W2.reshape(HID, F_IN, F_OUT).transpose(1, 2, 0).reshape(F_IN, F_OUT * HID)
    b2m = b2.reshape(F_IN, F_OUT)
    r_sel = jnp.asarray(np.kron(np.eye(F_OUT, dtype=np.float32),
                                np.ones((HID, 1), dtype=np.float32)))

    stats = _stats_call(edge_attr, W1)
    xj = _make_gather()(x, src_p)
    msg = _msg_call(stats, edge_attr, xj, W1, bn_gamma.reshape(1, HID),
                    bn_beta.reshape(1, HID), w2q.astype(jnp.bfloat16),
                    b2m.astype(jnp.bfloat16), r_sel)
    parts = _make_scatter()(msg, dst_p, jnp.zeros((NP, MSG_W), jnp.float32))
    x_p = jnp.pad(x, ((0, NP - N), (0, 0)))
    batch_p = jnp.pad(batch, (0, NP - N), constant_values=G)
    return _final_call(parts, x_p, root_W, bias.reshape(1, F_OUT),
                       batch_p.reshape(1, NP))
